# Initial kernel scaffold; baseline (speedup 1.0000x reference)
import functools

import jax
import jax.numpy as jnp
from jax import lax
from jax.experimental import pallas as pl
from jax.experimental.pallas import tpu as pltpu
from jax.experimental.pallas import tpu_sc as plsc

N = 10000
E = 160000
N_PAD = 10240
NT = 16
ROWS_T = N_PAD // NT
G = 128
NG = 79
E_PAD = NT * NG * G
H = 512
NUM_G = 8
PAD_NODE = N_PAD - 1

_MESH = plsc.VectorSubcoreMesh(core_axis_name="c", subcore_axis_name="s")


def _rsqrt_newton(d):
    i = plsc.bitcast(d, jnp.int32)
    y = plsc.bitcast(jnp.int32(0x5F3759DF) - lax.shift_right_logical(i, 1),
                     jnp.float32)
    for _ in range(3):
        y = y * (1.5 - 0.5 * d * y * y)
    return jnp.where(d > 0.0, y, 0.0)


def _prep_body(src_hbm, dst_hbm, ew_hbm, zeros16_hbm,
               dis_hbm, a_hbm,
               src_v, dst_v, ew_v, ewb, deg_loc, dis_loc, dis_full,
               deg16_sh, dis_sh):
    cid = lax.axis_index("c")
    sid = lax.axis_index("s")

    @pl.when(cid == 0)
    def _():
        pltpu.sync_copy(src_hbm.at[sid], src_v)
        pltpu.sync_copy(dst_hbm.at[sid], dst_v)
        pltpu.sync_copy(ew_hbm.at[sid], ew_v)
        pltpu.sync_copy(zeros16_hbm, deg16_sh.at[pl.ds(sid * ROWS_T, ROWS_T)])
        plsc.subcore_barrier()

        def g_body(g, _):
            def j_body(j, _):
                bc = plsc.load_gather(
                    ew_v, [jnp.full((16,), g, jnp.int32),
                           jnp.full((16,), j, jnp.int32)])
                ewb[j, :] = bc
                return 0
            lax.fori_loop(0, G, j_body, 0)
            pltpu.sync_copy(ewb, deg16_sh.at[dst_v.at[g]], add=True)
            return 0
        lax.fori_loop(0, NG, g_body, 0)
        plsc.subcore_barrier()

        pltpu.sync_copy(deg16_sh.at[pl.ds(sid * ROWS_T, ROWS_T)], deg_loc)

        def d_body(k, _):
            rows = lax.iota(jnp.int32, 16) + k * 16
            d = plsc.load_gather(deg_loc, [rows, jnp.zeros((16,), jnp.int32)])
            dis_loc[pl.ds(k * 16, 16)] = _rsqrt_newton(d)
            return 0
        lax.fori_loop(0, ROWS_T // 16, d_body, 0)
        pltpu.sync_copy(dis_loc, dis_sh.at[pl.ds(sid * ROWS_T, ROWS_T)])
        pltpu.sync_copy(dis_loc, dis_hbm.at[pl.ds(sid * ROWS_T, ROWS_T)])
        plsc.subcore_barrier()
        pltpu.sync_copy(dis_sh, dis_full)

        def a_body(g, _):
            for k in range(G // 16):
                sv = src_v[g, pl.ds(k * 16, 16)]
                dg = plsc.load_gather(dis_full, [sv])
                ew_v[g, pl.ds(k * 16, 16)] = ew_v[g, pl.ds(k * 16, 16)] * dg
            return 0
        lax.fori_loop(0, NG, a_body, 0)
        pltpu.sync_copy(ew_v, a_hbm.at[sid])


_prep = pl.kernel(
    _prep_body,
    out_type=(jax.ShapeDtypeStruct((N_PAD,), jnp.float32),
              jax.ShapeDtypeStruct((NT, NG, G), jnp.float32)),
    mesh=_MESH,
    scratch_types=[
        pltpu.VMEM((NG, G), jnp.int32),
        pltpu.VMEM((NG, G), jnp.int32),
        pltpu.VMEM((NG, G), jnp.float32),
        pltpu.VMEM((G, 16), jnp.float32),
        pltpu.VMEM((ROWS_T, 16), jnp.float32),
        pltpu.VMEM((ROWS_T,), jnp.float32),
        pltpu.VMEM((N_PAD,), jnp.float32),
        pltpu.VMEM_SHARED((N_PAD, 16), jnp.float32),
        pltpu.VMEM_SHARED((N_PAD,), jnp.float32),
    ],
)


def _edge_body(C, xc_hbm, src_hbm, dst_hbm, a_hbm, zrows_hbm,
               tx_hbm,
               src_v, dst_v, a_v, buf0, buf1, sem0, sem1, acc_sh):
    cid = lax.axis_index("c")
    sid = lax.axis_index("s")
    pltpu.sync_copy(src_hbm.at[sid], src_v)
    pltpu.sync_copy(dst_hbm.at[sid], dst_v)
    pltpu.sync_copy(a_hbm.at[sid], a_v)
    CH = C // 2

    for cc in range(CH):
        chunk = cid * CH + cc
        tab = xc_hbm.at[chunk]
        pltpu.sync_copy(zrows_hbm, acc_sh.at[pl.ds(sid * ROWS_T, ROWS_T)])
        plsc.subcore_barrier()

        def scale(g, buf):
            def j_body(j, _):
                ab = plsc.load_gather(
                    a_v, [jnp.full((16,), g, jnp.int32),
                          jnp.full((16,), j, jnp.int32)])
                for k in range(8):
                    buf[j, pl.ds(k * 16, 16)] = buf[j, pl.ds(k * 16, 16)] * ab
                return 0
            lax.fori_loop(0, G, j_body, 0)

        pltpu.async_copy(tab.at[src_v.at[0]], buf0, sem0)
        pltpu.async_copy(tab.at[src_v.at[1]], buf1, sem1)

        def i_body(i, _):
            g0 = 2 * i
            g1 = 2 * i + 1
            pltpu.make_async_copy(tab.at[src_v.at[g0]], buf0, sem0).wait()
            scale(g0, buf0)
            pltpu.sync_copy(buf0, acc_sh.at[dst_v.at[g0]], add=True)

            @pl.when(g0 + 2 < NG)
            def _():
                pltpu.async_copy(tab.at[src_v.at[g0 + 2]], buf0, sem0)

            @pl.when(g1 < NG)
            def _():
                pltpu.make_async_copy(tab.at[src_v.at[g1]], buf1, sem1).wait()
                scale(g1, buf1)
                pltpu.sync_copy(buf1, acc_sh.at[dst_v.at[g1]], add=True)

            @pl.when(g1 + 2 < NG)
            def _():
                pltpu.async_copy(tab.at[src_v.at[g1 + 2]], buf1, sem1)
            return 0
        lax.fori_loop(0, (NG + 1) // 2, i_body, 0)
        plsc.subcore_barrier()
        pltpu.sync_copy(acc_sh.at[pl.ds(sid * ROWS_T, ROWS_T)],
                        tx_hbm.at[chunk, pl.ds(sid * ROWS_T, ROWS_T)])
        plsc.subcore_barrier()


def _make_edge(C):
    return pl.kernel(
        functools.partial(_edge_body, C),
        out_type=jax.ShapeDtypeStruct((C, N_PAD, 128), jnp.float32),
        mesh=_MESH,
        scratch_types=[
            pltpu.VMEM((NG, G), jnp.int32),
            pltpu.VMEM((NG, G), jnp.int32),
            pltpu.VMEM((NG, G), jnp.float32),
            pltpu.VMEM((G, 128), jnp.float32),
            pltpu.VMEM((G, 128), jnp.float32),
            pltpu.SemaphoreType.DMA,
            pltpu.SemaphoreType.DMA,
            pltpu.VMEM_SHARED((N_PAD, 128), jnp.float32),
        ],
    )


_edge2 = _make_edge(2)
_edge4 = _make_edge(4)


def _conv_block(x_ref, tx_ref, dis_ref, w0_ref, w1_ref, b_ref, o_ref):
    nd = -dis_ref[...][:, 0:1]
    txs = tx_ref[...] * nd
    acc = jnp.dot(x_ref[...], w0_ref[...], preferred_element_type=jnp.float32)
    acc = acc + jnp.dot(txs, w1_ref[...], preferred_element_type=jnp.float32)
    acc = acc + b_ref[...]
    o_ref[...] = jnp.maximum(acc, 0.0)


def _conv(x, tx, dis128, w0, w1, b):
    fin = x.shape[1]
    grid = N_PAD // 256
    return pl.pallas_call(
        _conv_block,
        grid=(grid,),
        in_specs=[
            pl.BlockSpec((256, fin), lambda i: (i, 0)),
            pl.BlockSpec((256, fin), lambda i: (i, 0)),
            pl.BlockSpec((256, 128), lambda i: (i, 0)),
            pl.BlockSpec((fin, H), lambda i: (0, 0)),
            pl.BlockSpec((fin, H), lambda i: (0, 0)),
            pl.BlockSpec((1, H), lambda i: (0, 0)),
        ],
        out_specs=pl.BlockSpec((256, H), lambda i: (i, 0)),
        out_shape=jax.ShapeDtypeStruct((N_PAD, H), jnp.float32),
    )(x, tx, dis128, w0, w1, b.reshape(1, H))


def _convb_pool_block(x_ref, tx_ref, dis_ref, batch_ref, w0_ref, w1_ref,
                      b_ref, s_ref, c_ref):
    i = pl.program_id(0)
    nd = -dis_ref[...][:, 0:1]
    txs = tx_ref[...] * nd
    acc = jnp.dot(x_ref[...], w0_ref[...], preferred_element_type=jnp.float32)
    acc = acc + jnp.dot(txs, w1_ref[...], preferred_element_type=jnp.float32)
    h = jnp.maximum(acc + b_ref[...], 0.0)
    oh = (batch_ref[...][:, 0:1] ==
          lax.broadcasted_iota(jnp.int32, (1, NUM_G), 1)).astype(jnp.float32)
    ps = lax.dot_general(oh, h, (((0,), (0,)), ((), ())),
                         preferred_element_type=jnp.float32)
    cs = jnp.sum(oh, axis=0)

    @pl.when(i == 0)
    def _():
        s_ref[...] = jnp.zeros_like(s_ref)
        c_ref[...] = jnp.zeros_like(c_ref)

    s_ref[...] += ps
    c_ref[...] += jnp.broadcast_to(cs[:, None], (NUM_G, 128))


def _convb_pool(x, tx, dis128, batch128, w0, w1, b):
    fin = x.shape[1]
    grid = N_PAD // 256
    return pl.pallas_call(
        _convb_pool_block,
        grid=(grid,),
        in_specs=[
            pl.BlockSpec((256, fin), lambda i: (i, 0)),
            pl.BlockSpec((256, fin), lambda i: (i, 0)),
            pl.BlockSpec((256, 128), lambda i: (i, 0)),
            pl.BlockSpec((256, 128), lambda i: (i, 0)),
            pl.BlockSpec((fin, H), lambda i: (0, 0)),
            pl.BlockSpec((fin, H), lambda i: (0, 0)),
            pl.BlockSpec((1, H), lambda i: (0, 0)),
        ],
        out_specs=[
            pl.BlockSpec((NUM_G, H), lambda i: (0, 0)),
            pl.BlockSpec((NUM_G, 128), lambda i: (0, 0)),
        ],
        out_shape=[
            jax.ShapeDtypeStruct((NUM_G, H), jnp.float32),
            jax.ShapeDtypeStruct((NUM_G, 128), jnp.float32),
        ],
    )(x, tx, dis128, batch128, w0, w1, b.reshape(1, H))


def _final_block(s1_ref, c1_ref, s2_ref, c2_ref, w_ref, b_ref, o_ref):
    p1 = s1_ref[...] / jnp.maximum(c1_ref[...][:, 0:1], 1.0)
    p2 = s2_ref[...] / jnp.maximum(c2_ref[...][:, 0:1], 1.0)
    p = 0.5 * (p1 + p2)
    o_ref[...] = jnp.dot(p, w_ref[...],
                         preferred_element_type=jnp.float32) + b_ref[...]


def _final(s1, c1, s2, c2, fc_w, fc_b):
    return pl.pallas_call(
        _final_block,
        out_shape=jax.ShapeDtypeStruct((NUM_G, 128), jnp.float32),
    )(s1, c1, s2, c2, fc_w, fc_b.reshape(1, 128))


def _pad_edges(edge_index, ew):
    src = edge_index[0]
    dst = edge_index[1]
    pe = E_PAD - E
    src = jnp.concatenate([src, jnp.full((pe,), PAD_NODE, jnp.int32)])
    dst = jnp.concatenate([dst, jnp.full((pe,), PAD_NODE, jnp.int32)])
    ew = jnp.concatenate([ew, jnp.zeros((pe,), jnp.float32)])
    return (src.reshape(NT, NG, G), dst.reshape(NT, NG, G),
            ew.reshape(NT, NG, G))


def _chunked(x):
    f = x.shape[1]
    return x.reshape(N_PAD, f // 128, 128).transpose(1, 0, 2)


def _graph_tower(x, edge_index, ew, batch, w0a, w1a, ba, w0b, w1b, bb):
    fin = x.shape[1]
    srcr, dstr, ewr = _pad_edges(edge_index, ew)
    xp = jnp.pad(x, ((0, N_PAD - N), (0, 0)))
    zeros16 = jnp.zeros((ROWS_T, 16), jnp.float32)
    zrows = jnp.zeros((ROWS_T, 128), jnp.float32)

    dis, a = _prep(srcr, dstr, ewr, zeros16)
    dis128 = jnp.broadcast_to(dis[:, None], (N_PAD, 128))

    txa = _edge2(_chunked(xp), srcr, dstr, a, zrows)
    txa = txa.transpose(1, 0, 2).reshape(N_PAD, fin)
    h = _conv(xp, txa, dis128, w0a, w1a, ba)

    txb = _edge4(_chunked(h), srcr, dstr, a, zrows)
    txb = txb.transpose(1, 0, 2).reshape(N_PAD, H)

    batchp = jnp.concatenate([batch, jnp.full((N_PAD - N,), NUM_G, jnp.int32)])
    batch128 = jnp.broadcast_to(batchp[:, None], (N_PAD, 128))
    return _convb_pool(h, txb, dis128, batch128, w0b, w1b, bb)


def kernel(x1, edge_index1, edge_attr1, batch1, x2, edge_index2, edge_attr2,
           batch2, c1a_W0, c1a_W1, c1a_b, c1b_W0, c1b_W1, c1b_b,
           c2a_W0, c2a_W1, c2a_b, c2b_W0, c2b_W1, c2b_b, fc_W, fc_b):
    s1, c1 = _graph_tower(x1, edge_index1, edge_attr1, batch1,
                          c1a_W0, c1a_W1, c1a_b, c1b_W0, c1b_W1, c1b_b)
    s2, c2 = _graph_tower(x2, edge_index2, edge_attr2, batch2,
                          c2a_W0, c2a_W1, c2a_b, c2b_W0, c2b_W1, c2b_b)
    return _final(s1, c1, s2, c2, fc_W, fc_b)


# speedup vs baseline: 2.9036x; 2.9036x over previous
import functools

import jax
import jax.numpy as jnp
from jax import lax
from jax.experimental import pallas as pl
from jax.experimental.pallas import tpu as pltpu
from jax.experimental.pallas import tpu_sc as plsc

N = 10000
E = 160000
N_PAD = 10240
NT = 16
ROWS_T = N_PAD // NT
G = 128
NG = 80
NGP = NG + 2
E_PAD = NT * NG * G
H = 512
NUM_G = 8
PAD_NODE = N_PAD - 1

_MESH = plsc.VectorSubcoreMesh(core_axis_name="c", subcore_axis_name="s")


def _prep_body(dst_hbm, ew_hbm,
               deg_hbm,
               dst_v, ew_v, deg_loc, deg_sh):
    cid = lax.axis_index("c")
    sid = lax.axis_index("s")
    wid = cid * NT + sid
    half = ROWS_T // 2
    pltpu.sync_copy(dst_hbm.at[sid], dst_v)
    pltpu.sync_copy(ew_hbm.at[sid], ew_v)

    def z_body(k, _):
        deg_loc[pl.ds(k * 16, 16)] = jnp.zeros((16,), jnp.float32)
        return 0
    lax.fori_loop(0, ROWS_T // 16, z_body, 0)
    pltpu.sync_copy(deg_loc, deg_sh.at[pl.ds(sid * ROWS_T, ROWS_T)])
    plsc.subcore_barrier()

    def g_body(g, _):
        pltpu.sync_copy(ew_v.at[pl.ds(g * G, G)],
                        deg_sh.at[dst_v.at[g]], add=True)
        return 0
    lax.fori_loop(0, NG, g_body, 0)
    plsc.subcore_barrier()
    pltpu.sync_copy(deg_sh.at[pl.ds(wid * half, half)],
                    deg_loc.at[pl.ds(0, half)])
    pltpu.sync_copy(deg_loc.at[pl.ds(0, half)],
                    deg_hbm.at[pl.ds(wid * half, half)])


_prep = pl.kernel(
    _prep_body,
    out_type=jax.ShapeDtypeStruct((N_PAD,), jnp.float32),
    mesh=_MESH,
    scratch_types=[
        pltpu.VMEM((NGP, G), jnp.int32),
        pltpu.VMEM((NGP * G,), jnp.float32),
        pltpu.VMEM((ROWS_T,), jnp.float32),
        pltpu.VMEM_SHARED((N_PAD,), jnp.float32),
    ],
)


def _edge_body(C, xc_hbm, src_hbm, dst_hbm, a_hbm, zrows_hbm,
               tx_hbm,
               src_v, dring, aring, buf0, buf1,
               gsem0, gsem1, dsem0, dsem1, asem0, asem1, acc_sh):
    cid = lax.axis_index("c")
    sid = lax.axis_index("s")
    pltpu.sync_copy(src_hbm.at[sid], src_v)
    CH = C // 2

    dst_t = dst_hbm.at[sid]
    a_t = a_hbm.at[sid]

    for cc in range(CH):
        chunk = cid * CH + cc
        tab = xc_hbm.at[chunk]
        pltpu.sync_copy(zrows_hbm, acc_sh.at[pl.ds(sid * ROWS_T, ROWS_T)])
        plsc.subcore_barrier()

        def scale(b, buf):
            def jj_body(jj, _):
                a16 = aring[b, pl.ds(jj * 16, 16)]
                for l in range(16):
                    j = jj * 16 + l
                    ab = jnp.take(a16, jnp.full((16,), l, jnp.int32))
                    for k in range(8):
                        buf[j, pl.ds(k * 16, 16)] = (
                            buf[j, pl.ds(k * 16, 16)] * ab)
                return 0
            lax.fori_loop(0, G // 16, jj_body, 0)

        pltpu.async_copy(tab.at[src_v.at[0]], buf0, gsem0)
        pltpu.async_copy(tab.at[src_v.at[1]], buf1, gsem1)
        pltpu.async_copy(dst_t.at[0], dring.at[0], dsem0)
        pltpu.async_copy(dst_t.at[1], dring.at[1], dsem1)
        pltpu.async_copy(a_t.at[pl.ds(0, G)], aring.at[0], asem0)
        pltpu.async_copy(a_t.at[pl.ds(G, G)], aring.at[1], asem1)

        def half(g, b, buf, gsem, dsem, asem, dslot, aslot):
            del aslot
            pltpu.make_async_copy(tab.at[src_v.at[g]], buf, gsem).wait()
            pltpu.make_async_copy(a_t.at[pl.ds(0, G)], aring.at[b],
                                  asem).wait()
            scale(b, buf)
            pltpu.make_async_copy(dst_t.at[0], dslot, dsem).wait()
            pltpu.sync_copy(buf, acc_sh.at[dslot], add=True)
            pltpu.async_copy(tab.at[src_v.at[g + 2]], buf, gsem)
            pltpu.async_copy(dst_t.at[g + 2], dslot, dsem)
            pltpu.async_copy(a_t.at[pl.ds((g + 2) * G, G)], aring.at[b],
                             asem)

        def i_body(i, _):
            half(2 * i, 0, buf0, gsem0, dsem0, asem0, dring.at[0], None)
            half(2 * i + 1, 1, buf1, gsem1, dsem1, asem1, dring.at[1], None)
            return 0
        lax.fori_loop(0, NG // 2, i_body, 0)
        pltpu.make_async_copy(tab.at[src_v.at[0]], buf0, gsem0).wait()
        pltpu.make_async_copy(tab.at[src_v.at[1]], buf1, gsem1).wait()
        pltpu.make_async_copy(dst_t.at[0], dring.at[0], dsem0).wait()
        pltpu.make_async_copy(dst_t.at[1], dring.at[1], dsem1).wait()
        pltpu.make_async_copy(a_t.at[pl.ds(0, G)], aring.at[0], asem0).wait()
        pltpu.make_async_copy(a_t.at[pl.ds(0, G)], aring.at[1], asem1).wait()
        plsc.subcore_barrier()
        pltpu.sync_copy(acc_sh.at[pl.ds(sid * ROWS_T, ROWS_T)],
                        tx_hbm.at[chunk, pl.ds(sid * ROWS_T, ROWS_T)])
        plsc.subcore_barrier()


def _make_edge(C):
    return pl.kernel(
        functools.partial(_edge_body, C),
        out_type=jax.ShapeDtypeStruct((C, N_PAD, 128), jnp.float32),
        mesh=_MESH,
        scratch_types=[
            pltpu.VMEM((NGP, G), jnp.int32),
            pltpu.VMEM((2, G), jnp.int32),
            pltpu.VMEM((2, G), jnp.float32),
            pltpu.VMEM((G, 128), jnp.float32),
            pltpu.VMEM((G, 128), jnp.float32),
            pltpu.SemaphoreType.DMA,
            pltpu.SemaphoreType.DMA,
            pltpu.SemaphoreType.DMA,
            pltpu.SemaphoreType.DMA,
            pltpu.SemaphoreType.DMA,
            pltpu.SemaphoreType.DMA,
            pltpu.VMEM_SHARED((N_PAD, 128), jnp.float32),
        ],
    )


_edge2 = _make_edge(2)
_edge4 = _make_edge(4)


def _scale_block(x_ref, deg_ref, xs_ref, dis_ref):
    d = deg_ref[...][:, 0:1]
    dis = jnp.where(d > 0.0, lax.rsqrt(d), 0.0)
    xs_ref[...] = x_ref[...] * dis
    dis_ref[...] = jnp.broadcast_to(dis, dis_ref.shape)


def _scale(x, deg16):
    fin = x.shape[1]
    grid = N_PAD // 256
    return pl.pallas_call(
        _scale_block,
        grid=(grid,),
        in_specs=[
            pl.BlockSpec((256, fin), lambda i: (i, 0)),
            pl.BlockSpec((256, 16), lambda i: (i, 0)),
        ],
        out_specs=[
            pl.BlockSpec((256, fin), lambda i: (i, 0)),
            pl.BlockSpec((256, 128), lambda i: (i, 0)),
        ],
        out_shape=[
            jax.ShapeDtypeStruct((N_PAD, fin), jnp.float32),
            jax.ShapeDtypeStruct((N_PAD, 128), jnp.float32),
        ],
    )(x, deg16)


def _conv_block(x_ref, tx_ref, dis_ref, w0_ref, w1_ref, b_ref, o_ref, os_ref):
    nd = -dis_ref[...][:, 0:1]
    txs = (tx_ref[...] * nd).astype(jnp.bfloat16)
    xb = x_ref[...].astype(jnp.bfloat16)
    acc = jnp.dot(xb, w0_ref[...].astype(jnp.bfloat16),
                  preferred_element_type=jnp.float32)
    acc = acc + jnp.dot(txs, w1_ref[...].astype(jnp.bfloat16),
                        preferred_element_type=jnp.float32)
    acc = acc + b_ref[...]
    h = jnp.maximum(acc, 0.0)
    o_ref[...] = h
    os_ref[...] = h * (-nd)


def _conv(x, tx, dis128, w0, w1, b):
    fin = x.shape[1]
    grid = N_PAD // 256
    return pl.pallas_call(
        _conv_block,
        grid=(grid,),
        in_specs=[
            pl.BlockSpec((256, fin), lambda i: (i, 0)),
            pl.BlockSpec((256, fin), lambda i: (i, 0)),
            pl.BlockSpec((256, 128), lambda i: (i, 0)),
            pl.BlockSpec((fin, H), lambda i: (0, 0)),
            pl.BlockSpec((fin, H), lambda i: (0, 0)),
            pl.BlockSpec((1, H), lambda i: (0, 0)),
        ],
        out_specs=[
            pl.BlockSpec((256, H), lambda i: (i, 0)),
            pl.BlockSpec((256, H), lambda i: (i, 0)),
        ],
        out_shape=[
            jax.ShapeDtypeStruct((N_PAD, H), jnp.float32),
            jax.ShapeDtypeStruct((N_PAD, H), jnp.float32),
        ],
    )(x, tx, dis128, w0, w1, b.reshape(1, H))


def _convb_pool_block(x_ref, tx_ref, dis_ref, batch_ref, w0_ref, w1_ref,
                      b_ref, s_ref, c_ref):
    i = pl.program_id(0)
    nd = -dis_ref[...][:, 0:1]
    txs = (tx_ref[...] * nd).astype(jnp.bfloat16)
    xb = x_ref[...].astype(jnp.bfloat16)
    acc = jnp.dot(xb, w0_ref[...].astype(jnp.bfloat16),
                  preferred_element_type=jnp.float32)
    acc = acc + jnp.dot(txs, w1_ref[...].astype(jnp.bfloat16),
                        preferred_element_type=jnp.float32)
    h = jnp.maximum(acc + b_ref[...], 0.0)
    oh = (batch_ref[...][:, 0:1] ==
          lax.broadcasted_iota(jnp.int32, (1, NUM_G), 1)).astype(jnp.float32)
    ps = lax.dot_general(oh, h, (((0,), (0,)), ((), ())),
                         preferred_element_type=jnp.float32)
    cs = jnp.sum(oh, axis=0)

    @pl.when(i == 0)
    def _():
        s_ref[...] = jnp.zeros_like(s_ref)
        c_ref[...] = jnp.zeros_like(c_ref)

    s_ref[...] += ps
    c_ref[...] += jnp.broadcast_to(cs[:, None], (NUM_G, 128))


def _convb_pool(x, tx, dis128, batch128, w0, w1, b):
    fin = x.shape[1]
    grid = N_PAD // 256
    return pl.pallas_call(
        _convb_pool_block,
        grid=(grid,),
        in_specs=[
            pl.BlockSpec((256, fin), lambda i: (i, 0)),
            pl.BlockSpec((256, fin), lambda i: (i, 0)),
            pl.BlockSpec((256, 128), lambda i: (i, 0)),
            pl.BlockSpec((256, 128), lambda i: (i, 0)),
            pl.BlockSpec((fin, H), lambda i: (0, 0)),
            pl.BlockSpec((fin, H), lambda i: (0, 0)),
            pl.BlockSpec((1, H), lambda i: (0, 0)),
        ],
        out_specs=[
            pl.BlockSpec((NUM_G, H), lambda i: (0, 0)),
            pl.BlockSpec((NUM_G, 128), lambda i: (0, 0)),
        ],
        out_shape=[
            jax.ShapeDtypeStruct((NUM_G, H), jnp.float32),
            jax.ShapeDtypeStruct((NUM_G, 128), jnp.float32),
        ],
    )(x, tx, dis128, batch128, w0, w1, b.reshape(1, H))


def _final_block(s1_ref, c1_ref, s2_ref, c2_ref, w_ref, b_ref, o_ref):
    p1 = s1_ref[...] / jnp.maximum(c1_ref[...][:, 0:1], 1.0)
    p2 = s2_ref[...] / jnp.maximum(c2_ref[...][:, 0:1], 1.0)
    p = 0.5 * (p1 + p2)
    o_ref[...] = jnp.dot(p, w_ref[...],
                         preferred_element_type=jnp.float32) + b_ref[...]


def _final(s1, c1, s2, c2, fc_w, fc_b):
    return pl.pallas_call(
        _final_block,
        out_shape=jax.ShapeDtypeStruct((NUM_G, 128), jnp.float32),
    )(s1, c1, s2, c2, fc_w, fc_b.reshape(1, 128))


def _pad_edges(edge_index, ew):
    src = edge_index[0]
    dst = edge_index[1]
    pe = E_PAD - E
    src = jnp.concatenate([src, jnp.full((pe,), PAD_NODE, jnp.int32)])
    dst = jnp.concatenate([dst, jnp.full((pe,), PAD_NODE, jnp.int32)])
    ew = jnp.concatenate([ew, jnp.zeros((pe,), jnp.float32)])
    pad_i = jnp.full((NT, 2, G), PAD_NODE, jnp.int32)
    src = jnp.concatenate([src.reshape(NT, NG, G), pad_i], axis=1)
    dst = jnp.concatenate([dst.reshape(NT, NG, G), pad_i], axis=1)
    ew = jnp.concatenate([ew.reshape(NT, NG, G),
                          jnp.zeros((NT, 2, G), jnp.float32)], axis=1)
    return src, dst, ew.reshape(NT, NGP * G)


def _chunked(x):
    f = x.shape[1]
    return x.reshape(N_PAD, f // 128, 128).transpose(1, 0, 2)


def _graph_tower(x, edge_index, ew, batch, w0a, w1a, ba, w0b, w1b, bb):
    fin = x.shape[1]
    srcr, dstr, ewr = _pad_edges(edge_index, ew)
    xp = jnp.pad(x, ((0, N_PAD - N), (0, 0)))
    zrows = jnp.zeros((ROWS_T, 128), jnp.float32)

    deg = _prep(dstr, ewr)
    deg16 = jnp.broadcast_to(deg[:, None], (N_PAD, 16))
    xs, dis128 = _scale(xp, deg16)

    txa = _edge2(_chunked(xs), srcr, dstr, ewr, zrows)
    txa = txa.transpose(1, 0, 2).reshape(N_PAD, fin)
    h, hs = _conv(xp, txa, dis128, w0a, w1a, ba)

    txb = _edge4(_chunked(hs), srcr, dstr, ewr, zrows)
    txb = txb.transpose(1, 0, 2).reshape(N_PAD, H)

    batchp = jnp.concatenate([batch, jnp.full((N_PAD - N,), NUM_G, jnp.int32)])
    batch128 = jnp.broadcast_to(batchp[:, None], (N_PAD, 128))
    return _convb_pool(h, txb, dis128, batch128, w0b, w1b, bb)


def kernel(x1, edge_index1, edge_attr1, batch1, x2, edge_index2, edge_attr2,
           batch2, c1a_W0, c1a_W1, c1a_b, c1b_W0, c1b_W1, c1b_b,
           c2a_W0, c2a_W1, c2a_b, c2b_W0, c2b_W1, c2b_b, fc_W, fc_b):
    s1, c1 = _graph_tower(x1, edge_index1, edge_attr1, batch1,
                          c1a_W0, c1a_W1, c1a_b, c1b_W0, c1b_W1, c1b_b)
    s2, c2 = _graph_tower(x2, edge_index2, edge_attr2, batch2,
                          c2a_W0, c2a_W1, c2a_b, c2b_W0, c2b_W1, c2b_b)
    return _final(s1, c1, s2, c2, fc_W, fc_b)
